# baseline (speedup 1.0000x reference)
import jax
import jax.numpy as jnp
from jax.experimental import pallas as pl
from jax.experimental.pallas import tpu as pltpu

_T = 0.05
_BN = 5888
_NPAD = 100096


def _matmul_kernel(x_ref, memt_ref, out_ref):
    acc = jax.lax.dot_general(
        x_ref[...],
        memt_ref[...],
        dimension_numbers=(((1,), (0,)), ((), ())),
        preferred_element_type=jnp.float32,
    )
    out_ref[...] = acc / _T


@jax.jit
def kernel(x, memory):
    m, k = x.shape
    n = memory.shape[0]
    memt = memory.T
    grid = (_NPAD // _BN,)
    out = pl.pallas_call(
        _matmul_kernel,
        grid=grid,
        in_specs=[
            pl.BlockSpec((m, k), lambda i: (0, 0)),
            pl.BlockSpec((k, _BN), lambda i: (0, i)),
        ],
        out_specs=pl.BlockSpec((m, _BN), lambda i: (0, i)),
        out_shape=jax.ShapeDtypeStruct((m, _NPAD), jnp.float32),
        compiler_params=pltpu.CompilerParams(
            dimension_semantics=("arbitrary",),
            vmem_limit_bytes=63 * 1024 * 1024,
        ),
    )(x, memt)
    return out[:, :n]


# speedup vs baseline: 1.0008x; 1.0008x over previous
import jax
import jax.numpy as jnp
from jax.experimental import pallas as pl
from jax.experimental.pallas import tpu as pltpu

_T = 0.05
_BN = 4352


def _matmul_kernel(x_ref, memt_ref, out_ref):
    acc = jax.lax.dot_general(
        x_ref[...],
        memt_ref[...],
        dimension_numbers=(((1,), (0,)), ((), ())),
        preferred_element_type=jnp.float32,
    )
    out_ref[...] = acc / _T


@jax.jit
def kernel(x, memory):
    m, k = x.shape
    n = memory.shape[0]
    npad = -(-n // 128) * 128
    memt = memory.T
    grid = (pl.cdiv(npad, _BN),)
    out = pl.pallas_call(
        _matmul_kernel,
        grid=grid,
        in_specs=[
            pl.BlockSpec((m, k), lambda i: (0, 0)),
            pl.BlockSpec((k, _BN), lambda i: (0, i)),
        ],
        out_specs=pl.BlockSpec((m, _BN), lambda i: (0, i)),
        out_shape=jax.ShapeDtypeStruct((m, npad), jnp.float32),
        compiler_params=pltpu.CompilerParams(
            dimension_semantics=("arbitrary",),
            vmem_limit_bytes=63 * 1024 * 1024,
        ),
    )(x, memt)
    return out[:, :n]
